# Initial kernel scaffold; baseline (speedup 1.0000x reference)
import functools

import jax
import jax.numpy as jnp
from jax import lax
from jax.experimental import pallas as pl
from jax.experimental.pallas import tpu as pltpu
from jax.experimental.pallas import tpu_sc as plsc

_N_CORES = 2
_N_SUBCORES = 16
_N_TILES = _N_CORES * _N_SUBCORES
_K = 128


def _round_up(a, b):
    return (a + b - 1) // b * b


@functools.partial(jax.jit, static_argnums=(4, 5, 6))
def _sc_gather_scatter(table, src_idx, dst_idx, zeros, npad, n_chunks, d):
    rows_per_tile = npad // _N_SUBCORES
    mesh = plsc.VectorSubcoreMesh(core_axis_name="c", subcore_axis_name="s")

    @functools.partial(
        pl.kernel,
        out_type=jax.ShapeDtypeStruct((_N_CORES, npad, d), jnp.float32),
        mesh=mesh,
        scratch_types=[
            pltpu.VMEM((n_chunks, _K), jnp.int32),
            pltpu.VMEM((n_chunks, _K), jnp.int32),
            pltpu.VMEM((_K, d), jnp.float32),
            pltpu.VMEM((_K, d), jnp.float32),
            pltpu.VMEM_SHARED((npad, d), jnp.float32),
            pltpu.SemaphoreType.DMA,
            pltpu.SemaphoreType.DMA,
        ],
    )
    def body(table_hbm, src_hbm, dst_hbm, zeros_hbm, out_hbm,
             src_v, dst_v, buf0, buf1, acc, sem0, sem1):
        cid = lax.axis_index("c")
        sid = lax.axis_index("s")
        wid = cid * _N_SUBCORES + sid

        pltpu.sync_copy(src_hbm.at[wid], src_v)
        pltpu.sync_copy(dst_hbm.at[wid], dst_v)

        r0 = sid * rows_per_tile
        pltpu.sync_copy(zeros_hbm.at[pl.ds(r0, rows_per_tile)],
                        acc.at[pl.ds(r0, rows_per_tile)])
        plsc.subcore_barrier()

        n_pairs = n_chunks // 2

        pltpu.async_copy(table_hbm.at[src_v.at[0]], buf0, sem0)

        def pair(p, carry):
            j0 = 2 * p
            pltpu.async_copy(table_hbm.at[src_v.at[j0 + 1]], buf1, sem1)
            pltpu.make_async_copy(table_hbm.at[src_v.at[j0]], buf0, sem0).wait()
            pltpu.sync_copy(buf0, acc.at[dst_v.at[j0]], add=True)

            @pl.when(p + 1 < n_pairs)
            def _():
                pltpu.async_copy(table_hbm.at[src_v.at[j0 + 2]], buf0, sem0)

            pltpu.make_async_copy(table_hbm.at[src_v.at[j0 + 1]], buf1, sem1).wait()
            pltpu.sync_copy(buf1, acc.at[dst_v.at[j0 + 1]], add=True)
            return carry

        lax.fori_loop(0, n_pairs, pair, 0)

        plsc.subcore_barrier()
        pltpu.sync_copy(acc.at[pl.ds(r0, rows_per_tile)],
                        out_hbm.at[cid, pl.ds(r0, rows_per_tile)])

    return body(table, src_idx, dst_idx, zeros)


def _dis(degp):
    deg = degp[0, :, 0:1] + degp[1, :, 0:1]
    return jnp.where(deg > 0, lax.rsqrt(deg), 0.0)


def _tc_scale_matmul(xpad, w, degp, n):
    npad = xpad.shape[0]

    def body(x_ref, w_ref, degp_ref, y_ref):
        dis = _dis(degp_ref[...])
        xw = jnp.dot(x_ref[...], w_ref[...], preferred_element_type=jnp.float32)
        rows = lax.broadcasted_iota(jnp.int32, (npad, 1), 0)
        y_ref[...] = jnp.where(rows < n, dis * xw, 0.0)

    return pl.pallas_call(
        body,
        out_shape=jax.ShapeDtypeStruct((npad, w.shape[1]), jnp.float32),
    )(xpad, w, degp)


def _tc_combine_scale_matmul(part, degp, b, w, n):
    npad = part.shape[1]

    def body(p_ref, degp_ref, b_ref, w_ref, y_ref):
        dis = _dis(degp_ref[...])
        h = dis * (p_ref[0] + p_ref[1]) + b_ref[...][None, :]
        hw = jnp.dot(h, w_ref[...], preferred_element_type=jnp.float32)
        rows = lax.broadcasted_iota(jnp.int32, (npad, 1), 0)
        y_ref[...] = jnp.where(rows < n, dis * hw, 0.0)

    return pl.pallas_call(
        body,
        out_shape=jax.ShapeDtypeStruct((npad, w.shape[1]), jnp.float32),
    )(part, degp, b, w)


def _tc_combine_scale(part, degp, b):
    npad, d = part.shape[1], part.shape[2]

    def body(p_ref, degp_ref, b_ref, y_ref):
        dis = _dis(degp_ref[...])
        y_ref[...] = dis * (p_ref[0] + p_ref[1]) + b_ref[...][None, :]

    return pl.pallas_call(
        body,
        out_shape=jax.ShapeDtypeStruct((npad, d), jnp.float32),
    )(part, degp, b)


def kernel(x, edge_index, W1, b1, W2, b2):
    n, d_in = x.shape
    d = W1.shape[1]
    e = edge_index.shape[1]

    npad = _round_up(n + 1, _N_SUBCORES * 8)
    e_tot = e + n
    epad = _round_up(e_tot, _N_TILES * _K)
    n_chunks = epad // (_N_TILES * _K)

    ei = edge_index.astype(jnp.int32)
    loops = jnp.arange(n, dtype=jnp.int32)
    fill = jnp.full((epad - e_tot,), n, dtype=jnp.int32)
    src = jnp.concatenate([ei[0], loops, fill]).reshape(_N_TILES, n_chunks, _K)
    dst = jnp.concatenate([ei[1], loops, fill]).reshape(_N_TILES, n_chunks, _K)

    zeros_d = jnp.zeros((npad, d), jnp.float32)
    zeros_16 = jnp.zeros((npad, 16), jnp.float32)
    ones_16 = jnp.ones((npad, 16), jnp.float32)
    xpad = jnp.pad(x, ((0, npad - n), (0, 0)))

    degp = _sc_gather_scatter(ones_16, src, dst, zeros_16, npad, n_chunks, 16)

    y1 = _tc_scale_matmul(xpad, W1, degp, n)
    p1 = _sc_gather_scatter(y1, src, dst, zeros_d, npad, n_chunks, d)

    y2 = _tc_combine_scale_matmul(p1, degp, b1, W2, n)
    p2 = _sc_gather_scatter(y2, src, dst, zeros_d, npad, n_chunks, d)

    out = _tc_combine_scale(p2, degp, b2)
    return out[:n]


# speedup vs baseline: 13.7552x; 13.7552x over previous
import functools

import jax
import jax.numpy as jnp
from jax import lax
from jax.experimental import pallas as pl
from jax.experimental.pallas import tpu as pltpu
from jax.experimental.pallas import tpu_sc as plsc

_N_CORES = 2
_N_SUBCORES = 16
_K = 128


def _round_up(a, b):
    return (a + b - 1) // b * b


@functools.partial(jax.jit, static_argnums=(3, 4, 5))
def _sc_gather_scatter(table, src_idx, dst_idx, npad, n_chunks, d):
    half = npad // _N_CORES
    hacc = half + 512
    rows_wb = half // _N_SUBCORES
    mesh = plsc.VectorSubcoreMesh(core_axis_name="c", subcore_axis_name="s")

    @functools.partial(
        pl.kernel,
        out_type=jax.ShapeDtypeStruct((npad, d), jnp.float32),
        mesh=mesh,
        scratch_types=[
            pltpu.VMEM((n_chunks, _K), jnp.int32),
            pltpu.VMEM((n_chunks, _K), jnp.int32),
            pltpu.VMEM((_K, d), jnp.float32),
            pltpu.VMEM((_K, d), jnp.float32),
            pltpu.VMEM_SHARED((hacc, d), jnp.float32),
            pltpu.SemaphoreType.DMA,
            pltpu.SemaphoreType.DMA,
        ],
    )
    def body(table_hbm, src_hbm, dst_hbm, out_hbm,
             src_v, dst_v, b0, b1, acc, sem0, sem1):
        cid = lax.axis_index("c")
        sid = lax.axis_index("s")

        pltpu.sync_copy(src_hbm.at[sid], src_v)
        pltpu.sync_copy(dst_hbm.at[sid], dst_v)

        base = cid * half

        def remap(j, carry):
            for v in range(_K // 16):
                sl = pl.ds(v * 16, 16)
                raw = dst_v[j, sl]
                t = raw - base
                ok = (t >= 0) & (t < half)
                dst_v[j, sl] = jnp.where(ok, t, half + (raw & 511))
            return carry

        lax.fori_loop(0, n_chunks, remap, 0)

        def zrow(j, carry):
            for col in range(d // 16):
                b0[j, pl.ds(col * 16, 16)] = jnp.zeros((16,), jnp.float32)
            return carry

        lax.fori_loop(0, 64, zrow, 0)
        n_zb = hacc // 64
        for blk in range((n_zb + _N_SUBCORES - 1) // _N_SUBCORES):
            zi = blk * _N_SUBCORES + sid

            @pl.when(zi < n_zb)
            def _():
                pltpu.sync_copy(b0.at[pl.ds(0, 64)], acc.at[pl.ds(zi * 64, 64)])
        plsc.subcore_barrier()

        n_pairs = n_chunks // 2
        pltpu.async_copy(table_hbm.at[src_v.at[0]], b0, sem0)

        def pair(p, carry):
            j0 = 2 * p
            pltpu.async_copy(table_hbm.at[src_v.at[j0 + 1]], b1, sem1)
            pltpu.make_async_copy(table_hbm.at[src_v.at[j0]], b0, sem0).wait()
            pltpu.sync_copy(b0, acc.at[dst_v.at[j0]], add=True)

            @pl.when(p + 1 < n_pairs)
            def _():
                pltpu.async_copy(table_hbm.at[src_v.at[j0 + 2]], b0, sem0)

            pltpu.make_async_copy(table_hbm.at[src_v.at[j0 + 1]], b1,
                                  sem1).wait()
            pltpu.sync_copy(b1, acc.at[dst_v.at[j0 + 1]], add=True)
            return carry

        lax.fori_loop(0, n_pairs, pair, 0)

        plsc.subcore_barrier()
        for blk in range(rows_wb // 64):
            r0 = sid * rows_wb + blk * 64
            pltpu.sync_copy(acc.at[pl.ds(r0, 64)], b0.at[pl.ds(0, 64)])
            pltpu.sync_copy(b0.at[pl.ds(0, 64)],
                            out_hbm.at[pl.ds(base + r0, 64)])

    return body(table, src_idx, dst_idx)


@functools.partial(jax.jit, static_argnums=(1, 2))
def _sc_degree(dst_idx, npad, n_chunks):
    rows_per_tile = npad // _N_SUBCORES
    half_chunks = n_chunks // _N_CORES
    mesh = plsc.VectorSubcoreMesh(core_axis_name="c", subcore_axis_name="s")

    @functools.partial(
        pl.kernel,
        out_type=jax.ShapeDtypeStruct((_N_CORES * npad,), jnp.float32),
        mesh=mesh,
        scratch_types=[
            pltpu.VMEM((n_chunks, _K), jnp.int32),
            pltpu.VMEM((_K,), jnp.float32),
            pltpu.VMEM((rows_per_tile,), jnp.float32),
            pltpu.VMEM_SHARED((npad,), jnp.float32),
        ],
    )
    def body(dst_hbm, out_hbm, dst_v, ones_v, stage_v, acc):
        cid = lax.axis_index("c")
        sid = lax.axis_index("s")

        pltpu.sync_copy(dst_hbm.at[sid], dst_v)
        for i in range(_K // 16):
            ones_v[pl.ds(i * 16, 16)] = jnp.ones((16,), jnp.float32)

        def zfill(j, carry):
            stage_v[pl.ds(j * 16, 16)] = jnp.zeros((16,), jnp.float32)
            return carry

        lax.fori_loop(0, rows_per_tile // 16, zfill, 0)
        r0 = sid * rows_per_tile
        pltpu.sync_copy(stage_v, acc.at[pl.ds(r0, rows_per_tile)])
        plsc.subcore_barrier()

        def chunk(j, carry):
            pltpu.sync_copy(ones_v, acc.at[dst_v.at[j]], add=True)
            return carry

        lax.fori_loop(cid * half_chunks, (cid + 1) * half_chunks, chunk, 0)

        plsc.subcore_barrier()
        pltpu.sync_copy(acc.at[pl.ds(r0, rows_per_tile)], stage_v)
        pltpu.sync_copy(stage_v,
                        out_hbm.at[pl.ds(cid * npad + r0, rows_per_tile)])

    return body(dst_idx)


def _dis(degp):
    deg = degp[:, 0:1] + degp[:, 1:2]
    return jnp.where(deg > 0, lax.rsqrt(deg), 0.0)


def _tc_scale_matmul(xpad, w, degp, n):
    npad = xpad.shape[0]

    def body(x_ref, w_ref, degp_ref, y_ref):
        dis = _dis(degp_ref[...])
        xw = jnp.dot(x_ref[...], w_ref[...], preferred_element_type=jnp.float32)
        rows = lax.broadcasted_iota(jnp.int32, (npad, 1), 0)
        y_ref[...] = jnp.where(rows < n, dis * xw, 0.0)

    return pl.pallas_call(
        body,
        out_shape=jax.ShapeDtypeStruct((npad, w.shape[1]), jnp.float32),
    )(xpad, w, degp)


def _tc_combine_scale_matmul(s, degp, b, w, n):
    npad = s.shape[0]

    def body(s_ref, degp_ref, b_ref, w_ref, y_ref):
        dis = _dis(degp_ref[...])
        h = dis * s_ref[...] + b_ref[...][None, :]
        hw = jnp.dot(h, w_ref[...], preferred_element_type=jnp.float32)
        rows = lax.broadcasted_iota(jnp.int32, (npad, 1), 0)
        y_ref[...] = jnp.where(rows < n, dis * hw, 0.0)

    return pl.pallas_call(
        body,
        out_shape=jax.ShapeDtypeStruct((npad, w.shape[1]), jnp.float32),
    )(s, degp, b, w)


def _tc_combine_scale(s, degp, b):
    npad, d = s.shape

    def body(s_ref, degp_ref, b_ref, y_ref):
        dis = _dis(degp_ref[...])
        y_ref[...] = dis * s_ref[...] + b_ref[...][None, :]

    return pl.pallas_call(
        body,
        out_shape=jax.ShapeDtypeStruct((npad, d), jnp.float32),
    )(s, degp, b)


def kernel(x, edge_index, W1, b1, W2, b2):
    n, d_in = x.shape
    d = W1.shape[1]
    e = edge_index.shape[1]

    npad = _round_up(n + 1, _N_SUBCORES * 16)
    e_tot = e + n
    epad = _round_up(e_tot, _N_SUBCORES * _K * 2)
    n_chunks = epad // (_N_SUBCORES * _K)

    ei = edge_index.astype(jnp.int32)
    loops = jnp.arange(n, dtype=jnp.int32)
    fill = jnp.full((epad - e_tot,), n, dtype=jnp.int32)
    src_3d = jnp.concatenate([ei[0], loops, fill]).reshape(_N_SUBCORES, n_chunks, _K)
    dst_3d = jnp.concatenate([ei[1], loops, fill]).reshape(_N_SUBCORES, n_chunks, _K)

    xpad = jnp.pad(x, ((0, npad - n), (0, 0)))

    degp = _sc_degree(dst_3d, npad, n_chunks).reshape(_N_CORES, npad).T

    y1 = _tc_scale_matmul(xpad, W1, degp, n)
    s1 = _sc_gather_scatter(y1, src_3d, dst_3d, npad, n_chunks, d)

    y2 = _tc_combine_scale_matmul(s1, degp, b1, W2, n)
    s2 = _sc_gather_scatter(y2, src_3d, dst_3d, npad, n_chunks, d)

    out = _tc_combine_scale(s2, degp, b2)
    return out[:n]
